# Initial kernel scaffold; baseline (speedup 1.0000x reference)
import jax
import jax.numpy as jnp
from jax.experimental import pallas as pl


def kernel(x, embed_table, W, b, A, B_lora):
    raise NotImplementedError("write your pallas kernel here")


# speedup vs baseline: 1.7630x; 1.7630x over previous
import functools

import jax
import jax.numpy as jnp
from jax import lax
from jax.experimental import pallas as pl
from jax.experimental.pallas import tpu as pltpu
from jax.experimental.pallas import tpu_sc as plsc


_SC_CORES = 2
_SC_SUBCORES = 16


def _make_pool_kernel(V, D, B, L):
    nc, ns = _SC_CORES, _SC_SUBCORES
    nw = nc * ns
    assert B % nw == 0
    b_per_w = B // nw
    nvec = D // 16
    mesh = plsc.VectorSubcoreMesh(
        core_axis_name="c", subcore_axis_name="s", num_cores=nc, num_subcores=ns
    )

    @functools.partial(
        pl.kernel,
        out_type=jax.ShapeDtypeStruct((B, D), jnp.float32),
        mesh=mesh,
        scratch_types=[
            pltpu.VMEM((b_per_w * L,), jnp.int32),
            pltpu.VMEM((L, D), jnp.float32),
            pltpu.VMEM((b_per_w, D), jnp.float32),
            pltpu.SemaphoreType.DMA,
        ],
        compiler_params=pltpu.CompilerParams(use_tc_tiling_on_sc=False),
    )
    def pool(x_hbm, table_hbm, out_hbm, idx_v, rows_v, h_v, sem):
        wid = lax.axis_index("s") * nc + lax.axis_index("c")
        base = wid * b_per_w
        pltpu.sync_copy(x_hbm.at[pl.ds(base * L, b_per_w * L)], idx_v)

        def elem(e, carry):
            pltpu.async_copy(
                table_hbm.at[idx_v.at[pl.ds(e * L, L)]], rows_v, sem
            ).wait()

            def red(r, accs):
                return tuple(
                    accs[j] + rows_v[r, pl.ds(j * 16, 16)] for j in range(nvec)
                )

            accs = tuple(jnp.zeros((16,), jnp.float32) for _ in range(nvec))
            accs = lax.fori_loop(0, L, red, accs)
            scale = jnp.float32(1.0 / L)
            for j in range(nvec):
                h_v[e, pl.ds(j * 16, 16)] = accs[j] * scale
            return carry

        lax.fori_loop(0, b_per_w, elem, 0)
        pltpu.sync_copy(h_v, out_hbm.at[pl.ds(base, b_per_w)])

    return pool


def _make_proj_kernel(B, D, V, R, TV):
    nblk = pl.cdiv(V, TV)

    def body(h_ref, w_ref, b_ref, a_ref, bl_ref, out_ref):
        w_eff = w_ref[...] + jnp.dot(
            a_ref[...], bl_ref[...], preferred_element_type=jnp.float32
        )
        out_ref[...] = (
            jnp.dot(h_ref[...], w_eff, preferred_element_type=jnp.float32)
            + b_ref[...]
        )

    return pl.pallas_call(
        body,
        grid=(nblk,),
        in_specs=[
            pl.BlockSpec((B, D), lambda i: (0, 0)),
            pl.BlockSpec((D, TV), lambda i: (0, i)),
            pl.BlockSpec((1, TV), lambda i: (0, i)),
            pl.BlockSpec((D, R), lambda i: (0, 0)),
            pl.BlockSpec((R, TV), lambda i: (0, i)),
        ],
        out_specs=pl.BlockSpec((B, TV), lambda i: (0, i)),
        out_shape=jax.ShapeDtypeStruct((B, V), jnp.float32),
    )


@jax.jit
def kernel(x, embed_table, W, b, A, B_lora):
    B, L = x.shape
    V, D = embed_table.shape
    R = A.shape[1]

    pool = _make_pool_kernel(V, D, B, L)
    h = pool(x.reshape(-1), embed_table)

    proj = _make_proj_kernel(B, D, V, R, TV=1024)
    return proj(h, W, b.reshape(1, V), A, B_lora)


# speedup vs baseline: 1.8453x; 1.0466x over previous
import functools

import jax
import jax.numpy as jnp
from jax import lax
from jax.experimental import pallas as pl
from jax.experimental.pallas import tpu as pltpu
from jax.experimental.pallas import tpu_sc as plsc


_SC_CORES = 2
_SC_SUBCORES = 16


def _make_pool_kernel(V, D, B, L):
    nc, ns = _SC_CORES, _SC_SUBCORES
    nw = nc * ns
    assert B % nw == 0
    b_per_w = B // nw
    nvec = D // 16
    mesh = plsc.VectorSubcoreMesh(
        core_axis_name="c", subcore_axis_name="s", num_cores=nc, num_subcores=ns
    )

    @functools.partial(
        pl.kernel,
        out_type=jax.ShapeDtypeStruct((B, D), jnp.float32),
        mesh=mesh,
        scratch_types=[
            pltpu.VMEM((b_per_w * L,), jnp.int32),
            pltpu.VMEM((L, D), jnp.float32),
            pltpu.VMEM((b_per_w, D), jnp.float32),
            pltpu.SemaphoreType.DMA,
        ],
        compiler_params=pltpu.CompilerParams(use_tc_tiling_on_sc=False),
    )
    def pool(x_hbm, table_hbm, out_hbm, idx_v, rows_v, h_v, sem):
        wid = lax.axis_index("s") * nc + lax.axis_index("c")
        base = wid * b_per_w
        pltpu.sync_copy(x_hbm.at[pl.ds(base * L, b_per_w * L)], idx_v)

        def elem(e, carry):
            pltpu.async_copy(
                table_hbm.at[idx_v.at[pl.ds(e * L, L)]], rows_v, sem
            ).wait()

            def red(r, accs):
                return tuple(
                    accs[j] + rows_v[r, pl.ds(j * 16, 16)] for j in range(nvec)
                )

            accs = tuple(jnp.zeros((16,), jnp.float32) for _ in range(nvec))
            accs = lax.fori_loop(0, L, red, accs)
            scale = jnp.float32(1.0 / L)
            for j in range(nvec):
                h_v[e, pl.ds(j * 16, 16)] = accs[j] * scale
            return carry

        lax.fori_loop(0, b_per_w, elem, 0)
        pltpu.sync_copy(h_v, out_hbm.at[pl.ds(base, b_per_w)])

    return pool


def _make_proj_kernel(B, D, V, R, TV):
    nblk = pl.cdiv(V, TV)

    def body(h_ref, w_ref, b_ref, a_ref, bl_ref, out_ref):
        w_eff = w_ref[...] + jnp.dot(
            a_ref[...], bl_ref[...], preferred_element_type=jnp.float32
        )
        out_ref[...] = (
            jnp.dot(h_ref[...], w_eff, preferred_element_type=jnp.float32)
            + b_ref[...]
        )

    return pl.pallas_call(
        body,
        grid=(nblk,),
        in_specs=[
            pl.BlockSpec((B, D), lambda i: (0, 0)),
            pl.BlockSpec((D, TV), lambda i: (0, i)),
            pl.BlockSpec((1, TV), lambda i: (0, i)),
            pl.BlockSpec((D, R), lambda i: (0, 0)),
            pl.BlockSpec((R, TV), lambda i: (0, i)),
        ],
        out_specs=pl.BlockSpec((B, TV), lambda i: (0, i)),
        out_shape=jax.ShapeDtypeStruct((B, V), jnp.float32),
    )


@jax.jit
def kernel(x, embed_table, W, b, A, B_lora):
    B, L = x.shape
    V, D = embed_table.shape
    R = A.shape[1]

    pool = _make_pool_kernel(V, D, B, L)
    h = pool(x.reshape(-1), embed_table)

    proj = _make_proj_kernel(B, D, V, R, TV=2048)
    return proj(h, W, b.reshape(1, V), A, B_lora)
